# baseline (speedup 1.0000x reference)
import functools

import jax
import jax.numpy as jnp
from jax import lax
from jax.experimental import pallas as pl
from jax.experimental.pallas import tpu as pltpu
from jax.experimental.pallas import tpu_sc as plsc

_TB = 1152
_NH = 3


def _score_stage(r, cb_hi, cb2, k):
    bf16 = jnp.bfloat16
    s = jax.lax.dot_general(r.astype(bf16), cb_hi,
                            (((1,), (1,)), ((), ())),
                            preferred_element_type=jnp.float32)
    d = (jnp.sum(r * r, axis=1, keepdims=True) - 2.0 * s) + cb2
    return jnp.argmin(d, axis=1).astype(jnp.int32)


def _indices(r, cb, k):
    cb2 = jnp.sum(cb * cb, axis=1)[None, :]
    cb_hi = cb.astype(jnp.bfloat16)
    hh = r.shape[0] // _NH
    return jnp.concatenate(
        [_score_stage(r[j * hh:(j + 1) * hh], cb_hi, cb2, k)
         for j in range(_NH)], axis=0)


def _enc_body(k, x_ref, ew_ref, eb_ref, cb_ref, h_ref, idx_ref):
    bf16 = jnp.bfloat16
    h = jax.nn.gelu(
        jnp.dot(x_ref[...].astype(bf16), ew_ref[...].astype(bf16),
                preferred_element_type=jnp.float32)
        + eb_ref[...])
    h_ref[...] = h
    idx_ref[...] = _indices(h, cb_ref[...], k)[None, :]


def _mid_body(k, r_ref, q_ref, cb_ref, rn_ref, idx_ref, closs_ref):
    r = r_ref[...] - q_ref[...]
    rn_ref[...] = r
    idx_ref[...] = _indices(r, cb_ref[...], k)[None, :]
    acc = jnp.full((8, 128), jnp.sum(r * r), jnp.float32)

    @pl.when(pl.program_id(0) == 0)
    def _init():
        closs_ref[...] = acc

    @pl.when(pl.program_id(0) != 0)
    def _accum():
        closs_ref[...] += acc


def _dec_body(h_ref, r_ref, q_ref, dw_ref, db_ref, out_ref, closs_ref):
    bf16 = jnp.bfloat16
    r = r_ref[...] - q_ref[...]
    out_ref[...] = (jnp.dot((h_ref[...] - r).astype(bf16),
                            dw_ref[...].astype(bf16),
                            preferred_element_type=jnp.float32)
                    + db_ref[...])
    acc = jnp.full((8, 128), jnp.sum(r * r), jnp.float32)

    @pl.when(pl.program_id(0) == 0)
    def _init():
        closs_ref[...] = acc

    @pl.when(pl.program_id(0) != 0)
    def _accum():
        closs_ref[...] += acc


def _make_sc_gather(v, dd, n):
    info = plsc.get_sparse_core_info()
    nc, ns = info.num_cores, info.num_subcores
    nw = nc * ns
    b_per_w = n // nw
    mesh = plsc.VectorSubcoreMesh(core_axis_name="c", subcore_axis_name="s")

    @functools.partial(
        pl.kernel, mesh=mesh,
        out_type=jax.ShapeDtypeStruct((n, dd), jnp.float32),
        scratch_types=[
            pltpu.VMEM((b_per_w,), jnp.int32),
            pltpu.VMEM((b_per_w, dd), jnp.float32),
            pltpu.SemaphoreType.DMA,
        ],
    )
    def gather(table_hbm, idx_hbm, out_hbm, idx_v, rows_v, sem):
        wid = lax.axis_index("s") * nc + lax.axis_index("c")
        base = wid * b_per_w
        pltpu.sync_copy(idx_hbm.at[pl.ds(base, b_per_w)], idx_v)
        pltpu.async_copy(table_hbm.at[idx_v], rows_v, sem).wait()
        pltpu.sync_copy(rows_v, out_hbm.at[pl.ds(base, b_per_w)])

    return gather


def kernel(data_object, enc_W, enc_b, codebooks, dec_W, dec_b):
    b, t, c = data_object.shape
    nq, k, d = codebooks.shape
    n = b * t
    grid = n // _TB
    x = data_object.reshape(n, c)

    row_spec = pl.BlockSpec((_TB, d), lambda i: (i, 0))
    idx_spec = pl.BlockSpec((1, _TB), lambda i: (0, i))
    cb_spec = pl.BlockSpec((k, d), lambda i: (0, 0))
    closs_spec = pl.BlockSpec((8, 128), lambda i: (0, 0))
    closs_shape = jax.ShapeDtypeStruct((8, 128), jnp.float32)
    idx_shape = jax.ShapeDtypeStruct((1, n), jnp.int32)
    row_shape = jax.ShapeDtypeStruct((n, d), jnp.float32)

    sc_gather = _make_sc_gather(k, d, n)

    h, idx0 = pl.pallas_call(
        functools.partial(_enc_body, k),
        grid=(grid,),
        in_specs=[
            pl.BlockSpec((_TB, c), lambda i: (i, 0)),
            pl.BlockSpec((c, d), lambda i: (0, 0)),
            pl.BlockSpec((1, d), lambda i: (0, 0)),
            cb_spec,
        ],
        out_specs=[row_spec, idx_spec],
        out_shape=[row_shape, idx_shape],
    )(x, enc_W, enc_b.reshape(1, d), codebooks[0])

    idxs = [idx0]
    csums = []
    r = h
    for i in range(1, nq):
        q = sc_gather(codebooks[i - 1], idxs[-1].reshape(n))
        r, idx_i, csum_i = pl.pallas_call(
            functools.partial(_mid_body, k),
            grid=(grid,),
            in_specs=[row_spec, row_spec, cb_spec],
            out_specs=[row_spec, idx_spec, closs_spec],
            out_shape=[row_shape, idx_shape, closs_shape],
        )(r, q, codebooks[i])
        idxs.append(idx_i)
        csums.append(csum_i)

    q = sc_gather(codebooks[nq - 1], idxs[-1].reshape(n))
    logits, csum_f = pl.pallas_call(
        _dec_body,
        grid=(grid,),
        in_specs=[
            row_spec, row_spec, row_spec,
            pl.BlockSpec((d, c), lambda i: (0, 0)),
            pl.BlockSpec((1, c), lambda i: (0, 0)),
        ],
        out_specs=[pl.BlockSpec((_TB, c), lambda i: (i, 0)), closs_spec],
        out_shape=[jax.ShapeDtypeStruct((n, c), jnp.float32), closs_shape],
    )(h, r, q, dec_W, dec_b.reshape(1, c))
    csums.append(csum_f)

    closs = sum(cs[0, 0] for cs in csums) * (1.25 / (n * d))
    indices = jnp.stack([ix.reshape(b, t) for ix in idxs], axis=-1)
    return logits.reshape(b, t, c), closs, indices


# speedup vs baseline: 1.9828x; 1.9828x over previous
import functools

import jax
import jax.numpy as jnp
from jax.experimental import pallas as pl

_TB = 1152
_NH = 3


def _stage(r, rsum, cb_hi, cb_mid, cb_lo, cb2, k):
    bf16 = jnp.bfloat16
    s = jax.lax.dot_general(r.astype(bf16), cb_hi,
                            (((1,), (1,)), ((), ())),
                            preferred_element_type=jnp.float32)
    d = (rsum - 2.0 * s) + cb2
    idx = jnp.argmin(d, axis=1).astype(jnp.int32)
    oh = (jax.lax.broadcasted_iota(jnp.int32, (r.shape[0], k), 1)
          == idx[:, None]).astype(bf16)
    q = (jnp.dot(oh, cb_hi, preferred_element_type=jnp.float32)
         + jnp.dot(oh, cb_mid, preferred_element_type=jnp.float32)
         + jnp.dot(oh, cb_lo, preferred_element_type=jnp.float32))
    rn = r - q
    rsum_n = jnp.sum(rn * rn, axis=1, keepdims=True)
    return rn, rsum_n, idx


def _body(nq, k, x_ref, ew_ref, eb_ref, cb_ref, dw_ref, db_ref,
          out_ref, idx_ref, closs_ref):
    bf16 = jnp.bfloat16
    x = x_ref[...]
    h = jax.nn.gelu(
        jnp.dot(x.astype(bf16), ew_ref[...].astype(bf16),
                preferred_element_type=jnp.float32)
        + eb_ref[...])
    hh = _TB // _NH
    rs = [h[j * hh:(j + 1) * hh] for j in range(_NH)]
    rsums = [jnp.sum(rj * rj, axis=1, keepdims=True) for rj in rs]
    csum = jnp.float32(0.0)
    idxs = [[] for _ in range(_NH)]
    for i in range(nq):
        cb = cb_ref[i]
        cb2 = jnp.sum(cb * cb, axis=1)[None, :]
        cb_hi = cb.astype(bf16)
        res1 = cb - cb_hi.astype(jnp.float32)
        cb_mid = res1.astype(bf16)
        cb_lo = (res1 - cb_mid.astype(jnp.float32)).astype(bf16)
        for j in range(_NH):
            rs[j], rsums[j], idx = _stage(rs[j], rsums[j], cb_hi, cb_mid,
                                          cb_lo, cb2, k)
            idxs[j].append(idx)
            csum = csum + jnp.sum(rsums[j])
    r = jnp.concatenate(rs, axis=0)
    out_ref[...] = (jnp.dot((h - r).astype(bf16), dw_ref[...].astype(bf16),
                            preferred_element_type=jnp.float32)
                    + db_ref[...])
    idx_ref[...] = jnp.concatenate(
        [jnp.stack(ix, axis=1) for ix in idxs], axis=0)
    acc = jnp.full((8, 128), csum, jnp.float32)

    @pl.when(pl.program_id(0) == 0)
    def _init():
        closs_ref[...] = acc

    @pl.when(pl.program_id(0) != 0)
    def _accum():
        closs_ref[...] += acc


def kernel(data_object, enc_W, enc_b, codebooks, dec_W, dec_b):
    b, t, c = data_object.shape
    nq, k, d = codebooks.shape
    n = b * t
    grid = n // _TB
    x = data_object.reshape(n, c)

    out, idx, closs_acc = pl.pallas_call(
        functools.partial(_body, nq, k),
        grid=(grid,),
        in_specs=[
            pl.BlockSpec((_TB, c), lambda i: (i, 0)),
            pl.BlockSpec((c, d), lambda i: (0, 0)),
            pl.BlockSpec((1, d), lambda i: (0, 0)),
            pl.BlockSpec((nq, k, d), lambda i: (0, 0, 0)),
            pl.BlockSpec((d, c), lambda i: (0, 0)),
            pl.BlockSpec((1, c), lambda i: (0, 0)),
        ],
        out_specs=[
            pl.BlockSpec((_TB, c), lambda i: (i, 0)),
            pl.BlockSpec((_TB, nq), lambda i: (i, 0)),
            pl.BlockSpec((8, 128), lambda i: (0, 0)),
        ],
        out_shape=[
            jax.ShapeDtypeStruct((n, c), jnp.float32),
            jax.ShapeDtypeStruct((n, nq), jnp.int32),
            jax.ShapeDtypeStruct((8, 128), jnp.float32),
        ],
    )(x, enc_W, enc_b.reshape(1, d), codebooks, dec_W, dec_b.reshape(1, c))

    logits = out.reshape(b, t, c)
    closs = closs_acc[0, 0] * (1.25 / (n * d))
    return logits, closs, idx.reshape(b, t, nq)


# speedup vs baseline: 1.9913x; 1.0043x over previous
import functools

import jax
import jax.numpy as jnp
from jax.experimental import pallas as pl

_TB = 1152
_NH = 3


def _stage(r, cb_hi, cb_mid, cb_lo, cb2, k):
    bf16 = jnp.bfloat16
    s = jax.lax.dot_general(r.astype(bf16), cb_hi,
                            (((1,), (1,)), ((), ())),
                            preferred_element_type=jnp.float32)
    d = (jnp.sum(r * r, axis=1, keepdims=True) - 2.0 * s) + cb2
    idx = jnp.argmin(d, axis=1).astype(jnp.int32)
    oh = (jax.lax.broadcasted_iota(jnp.int32, (r.shape[0], k), 1)
          == idx[:, None]).astype(bf16)
    q = (jnp.dot(oh, cb_hi, preferred_element_type=jnp.float32)
         + jnp.dot(oh, cb_mid, preferred_element_type=jnp.float32)
         + jnp.dot(oh, cb_lo, preferred_element_type=jnp.float32))
    return r - q, idx


def _body(nq, k, x_ref, ew_ref, eb_ref, cb_ref, dw_ref, db_ref,
          out_ref, idx_ref, closs_ref):
    bf16 = jnp.bfloat16
    x = x_ref[...]
    h = jax.nn.gelu(
        jnp.dot(x.astype(bf16), ew_ref[...].astype(bf16),
                preferred_element_type=jnp.float32)
        + eb_ref[...])
    hh = _TB // _NH
    rs = [h[j * hh:(j + 1) * hh] for j in range(_NH)]
    csum = jnp.float32(0.0)
    idxs = [[] for _ in range(_NH)]
    for i in range(nq):
        cb = cb_ref[i]
        cb2 = jnp.sum(cb * cb, axis=1)[None, :]
        cb_hi = cb.astype(bf16)
        res1 = cb - cb_hi.astype(jnp.float32)
        cb_mid = res1.astype(bf16)
        cb_lo = (res1 - cb_mid.astype(jnp.float32)).astype(bf16)
        for j in range(_NH):
            rs[j], idx = _stage(rs[j], cb_hi, cb_mid, cb_lo, cb2, k)
            idxs[j].append(idx)
        for j in range(_NH):
            csum = csum + jnp.sum(rs[j] * rs[j])
    r = jnp.concatenate(rs, axis=0)
    out_ref[...] = (jnp.dot((h - r).astype(bf16), dw_ref[...].astype(bf16),
                            preferred_element_type=jnp.float32)
                    + db_ref[...])
    idx_ref[...] = jnp.concatenate(
        [jnp.stack(ix, axis=1) for ix in idxs], axis=0)
    acc = jnp.full((8, 128), csum, jnp.float32)

    @pl.when(pl.program_id(0) == 0)
    def _init():
        closs_ref[...] = acc

    @pl.when(pl.program_id(0) != 0)
    def _accum():
        closs_ref[...] += acc


def kernel(data_object, enc_W, enc_b, codebooks, dec_W, dec_b):
    b, t, c = data_object.shape
    nq, k, d = codebooks.shape
    n = b * t
    grid = n // _TB
    x = data_object.reshape(n, c)

    out, idx, closs_acc = pl.pallas_call(
        functools.partial(_body, nq, k),
        grid=(grid,),
        in_specs=[
            pl.BlockSpec((_TB, c), lambda i: (i, 0)),
            pl.BlockSpec((c, d), lambda i: (0, 0)),
            pl.BlockSpec((1, d), lambda i: (0, 0)),
            pl.BlockSpec((nq, k, d), lambda i: (0, 0, 0)),
            pl.BlockSpec((d, c), lambda i: (0, 0)),
            pl.BlockSpec((1, c), lambda i: (0, 0)),
        ],
        out_specs=[
            pl.BlockSpec((_TB, c), lambda i: (i, 0)),
            pl.BlockSpec((_TB, nq), lambda i: (i, 0)),
            pl.BlockSpec((8, 128), lambda i: (0, 0)),
        ],
        out_shape=[
            jax.ShapeDtypeStruct((n, c), jnp.float32),
            jax.ShapeDtypeStruct((n, nq), jnp.int32),
            jax.ShapeDtypeStruct((8, 128), jnp.float32),
        ],
    )(x, enc_W, enc_b.reshape(1, d), codebooks, dec_W, dec_b.reshape(1, c))

    logits = out.reshape(b, t, c)
    closs = closs_acc[0, 0] * (1.25 / (n * d))
    return logits, closs, idx.reshape(b, t, nq)
